# baseline (device time: 53934 ns/iter reference)
import jax
import jax.numpy as jnp
from jax import lax
from jax.experimental import pallas as pl
from jax.experimental.pallas import tpu as pltpu

N_DEV = 8
N_PARTS = 3
N_SUB = 2
MASKS = (1, 3, 4)


def kernel(A, B):
    m, k = A.shape
    _, n = B.shape
    part = m // N_PARTS
    sub = n // N_SUB
    q1 = part // 2
    q2 = part // 4

    def body(a_ref, b_ref, out_ref, partial_ref, rs_send, rs_recv, ag_buf,
             rs_send_sems, rs_recv_sems, ag_send_sems, ag_recv_sems):
        my = lax.axis_index("i")
        sides = (
            jnp.bitwise_and(jnp.bitwise_xor(my, my // 2), 1),
            jnp.bitwise_and(my // 2, 1),
            jnp.bitwise_and(my // 4, 1),
        )
        partners = tuple(jnp.bitwise_xor(my, msk) for msk in MASKS)

        barrier_sem = pltpu.get_barrier_semaphore()
        for pt in partners:
            pl.semaphore_signal(
                barrier_sem, inc=1,
                device_id=(pt,), device_id_type=pl.DeviceIdType.MESH,
            )

        all_rdmas = []

        def start_rs(p, c, j, rows, d):
            r = pltpu.make_async_remote_copy(
                src_ref=rs_send.at[p, c, j, pl.ds(0, rows), :],
                dst_ref=rs_recv.at[p, c, j, pl.ds(0, rows), :],
                send_sem=rs_send_sems.at[p, c, j],
                recv_sem=rs_recv_sems.at[p, c, j],
                device_id=(partners[d],),
                device_id_type=pl.DeviceIdType.MESH,
            )
            r.start()
            all_rdmas.append(r)
            return r

        def start_ag(p, c, ja, row_off, rows, d):
            r = pltpu.make_async_remote_copy(
                src_ref=ag_buf.at[p, c, pl.ds(row_off, rows), :],
                dst_ref=ag_buf.at[p, c, pl.ds(row_off, rows), :],
                send_sem=ag_send_sems.at[p, c, ja],
                recv_sem=ag_recv_sems.at[p, c, ja],
                device_id=(partners[d],),
                device_id_type=pl.DeviceIdType.MESH,
            )
            r.start()
            all_rdmas.append(r)
            return r

        dims = [[(p + j) % 3 for j in range(3)] for p in range(N_PARTS)]
        kr0 = [None] * N_PARTS
        sr0 = [None] * N_PARTS
        kr1 = [None] * N_PARTS
        sr1 = [None] * N_PARTS
        for p in range(N_PARTS):
            s0, s1 = sides[dims[p][0]], sides[dims[p][1]]
            kr0[p] = s0 * q1
            sr0[p] = (1 - s0) * q1
            kr1[p] = kr0[p] + s1 * q2
            sr1[p] = kr0[p] + (1 - s1) * q2

        partial_ref[...] = jnp.dot(
            a_ref[...], b_ref[...], preferred_element_type=jnp.float32
        )
        for p in range(N_PARTS):
            base = p * part
            for c in range(N_SUB):
                rs_send[p, c, 0, :q1, :] = partial_ref[
                    pl.ds(base + sr0[p], q1), pl.ds(c * sub, sub)
                ].astype(jnp.bfloat16)

        pl.semaphore_wait(barrier_sem, 3)

        rd = [[None] * N_SUB for _ in range(N_PARTS)]
        for p in range(N_PARTS):
            for c in range(N_SUB):
                rd[p][c] = start_rs(p, c, 0, q1, dims[p][0])

        for c in range(N_SUB):
            for p in range(N_PARTS):
                base = p * part
                cols = pl.ds(c * sub, sub)
                rd[p][c].wait_recv()
                o_s = sr1[p] - kr0[p]
                o_k = kr1[p] - kr0[p]
                rs_send[p, c, 1, :q2, :] = (
                    partial_ref[pl.ds(base + sr1[p], q2), cols]
                    + rs_recv[p, c, 0, pl.ds(o_s, q2), :].astype(jnp.float32)
                ).astype(jnp.bfloat16)
                rd[p][c] = start_rs(p, c, 1, q2, dims[p][1])
                partial_ref[pl.ds(base + kr1[p], q2), cols] = (
                    partial_ref[pl.ds(base + kr1[p], q2), cols]
                    + rs_recv[p, c, 0, pl.ds(o_k, q2), :].astype(jnp.float32)
                )

        for c in range(N_SUB):
            for p in range(N_PARTS):
                base = p * part
                cols = pl.ds(c * sub, sub)
                rd[p][c].wait_recv()
                acc = (
                    partial_ref[pl.ds(base + kr1[p], q2), cols]
                    + rs_recv[p, c, 1, pl.ds(0, q2), :].astype(jnp.float32)
                )
                rs_send[p, c, 2, :q2, :] = acc.astype(jnp.bfloat16)
                rd[p][c] = start_rs(p, c, 2, q2, dims[p][2])
                partial_ref[pl.ds(base + kr1[p], q2), cols] = acc

        for c in range(N_SUB):
            for p in range(N_PARTS):
                base = p * part
                cols = pl.ds(c * sub, sub)
                rd[p][c].wait_recv()
                chunk = jnp.maximum(
                    partial_ref[pl.ds(base + kr1[p], q2), cols]
                    + rs_recv[p, c, 2, pl.ds(0, q2), :].astype(jnp.float32),
                    0.0,
                )
                out_ref[pl.ds(base + kr1[p], q2), cols] = chunk
                ag_buf[p, c, pl.ds(kr1[p], q2), :] = chunk.astype(jnp.bfloat16)
                rd[p][c] = start_ag(p, c, 0, kr1[p], q2, dims[p][1])

        for c in range(N_SUB):
            for p in range(N_PARTS):
                base = p * part
                cols = pl.ds(c * sub, sub)
                rd[p][c].wait_recv()
                rd[p][c] = start_ag(p, c, 1, kr0[p], q1, dims[p][0])
                out_ref[pl.ds(base + sr1[p], q2), cols] = ag_buf[
                    p, c, pl.ds(sr1[p], q2), :
                ].astype(jnp.float32)

        for c in range(N_SUB):
            for p in range(N_PARTS):
                base = p * part
                cols = pl.ds(c * sub, sub)
                rd[p][c].wait_recv()
                out_ref[pl.ds(base + sr0[p], q1), cols] = ag_buf[
                    p, c, pl.ds(sr0[p], q1), :
                ].astype(jnp.float32)

        for r in all_rdmas:
            r.wait_send()

    return pl.pallas_call(
        body,
        out_shape=jax.ShapeDtypeStruct((m, n), jnp.float32),
        in_specs=[
            pl.BlockSpec(memory_space=pltpu.VMEM),
            pl.BlockSpec(memory_space=pltpu.VMEM),
        ],
        out_specs=pl.BlockSpec(memory_space=pltpu.VMEM),
        scratch_shapes=[
            pltpu.VMEM((m, n), jnp.float32),
            pltpu.VMEM((N_PARTS, N_SUB, 3, q1, sub), jnp.bfloat16),
            pltpu.VMEM((N_PARTS, N_SUB, 3, q1, sub), jnp.bfloat16),
            pltpu.VMEM((N_PARTS, N_SUB, part, sub), jnp.bfloat16),
            pltpu.SemaphoreType.DMA((N_PARTS, N_SUB, 3)),
            pltpu.SemaphoreType.DMA((N_PARTS, N_SUB, 3)),
            pltpu.SemaphoreType.DMA((N_PARTS, N_SUB, 2)),
            pltpu.SemaphoreType.DMA((N_PARTS, N_SUB, 2)),
        ],
        compiler_params=pltpu.CompilerParams(
            collective_id=0,
            vmem_limit_bytes=100 * 1024 * 1024,
        ),
    )(A, B)


# device time: 52841 ns/iter; 1.0207x vs baseline; 1.0207x over previous
import jax
import jax.numpy as jnp
from jax import lax
from jax.experimental import pallas as pl
from jax.experimental.pallas import tpu as pltpu

N_DEV = 8
N_PARTS = 3
N_SUB = 2
MASKS = (1, 3, 4)


def kernel(A, B):
    m, k = A.shape
    _, n = B.shape
    part = m // N_PARTS
    sub = n // N_SUB
    q1 = part // 2
    q2 = part // 4

    def body(a_ref, b_ref, out_ref, partial_ref, rs_recv, ag_buf,
             rs_send_sems, rs_recv_sems, ag_send_sems, ag_recv_sems):
        my = lax.axis_index("i")
        sides = (
            jnp.bitwise_and(jnp.bitwise_xor(my, my // 2), 1),
            jnp.bitwise_and(my // 2, 1),
            jnp.bitwise_and(my // 4, 1),
        )
        partners = tuple(jnp.bitwise_xor(my, msk) for msk in MASKS)

        barrier_sem = pltpu.get_barrier_semaphore()
        for pt in partners:
            pl.semaphore_signal(
                barrier_sem, inc=1,
                device_id=(pt,), device_id_type=pl.DeviceIdType.MESH,
            )

        all_rdmas = []

        def start_rs(src_ref, p, c, j, rows, d):
            r = pltpu.make_async_remote_copy(
                src_ref=src_ref,
                dst_ref=rs_recv.at[p, c, j, pl.ds(0, rows), :],
                send_sem=rs_send_sems.at[p, c, j],
                recv_sem=rs_recv_sems.at[p, c, j],
                device_id=(partners[d],),
                device_id_type=pl.DeviceIdType.MESH,
            )
            r.start()
            all_rdmas.append(r)
            return r

        def start_ag(p, c, ja, row_off, rows, d):
            r = pltpu.make_async_remote_copy(
                src_ref=ag_buf.at[p, c, pl.ds(row_off, rows), :],
                dst_ref=ag_buf.at[p, c, pl.ds(row_off, rows), :],
                send_sem=ag_send_sems.at[p, c, ja],
                recv_sem=ag_recv_sems.at[p, c, ja],
                device_id=(partners[d],),
                device_id_type=pl.DeviceIdType.MESH,
            )
            r.start()
            all_rdmas.append(r)
            return r

        dims = [[(p + j) % 3 for j in range(3)] for p in range(N_PARTS)]
        kr0 = [None] * N_PARTS
        sr0 = [None] * N_PARTS
        kr1 = [None] * N_PARTS
        sr1 = [None] * N_PARTS
        for p in range(N_PARTS):
            s0, s1 = sides[dims[p][0]], sides[dims[p][1]]
            kr0[p] = s0 * q1
            sr0[p] = (1 - s0) * q1
            kr1[p] = kr0[p] + s1 * q2
            sr1[p] = kr0[p] + (1 - s1) * q2

        partial_ref[...] = jnp.dot(
            a_ref[...], b_ref[...], preferred_element_type=jnp.float32
        ).astype(jnp.bfloat16)

        pl.semaphore_wait(barrier_sem, 3)

        rd = [[None] * N_SUB for _ in range(N_PARTS)]
        for p in range(N_PARTS):
            base = p * part
            for c in range(N_SUB):
                rd[p][c] = start_rs(
                    partial_ref.at[pl.ds(base + sr0[p], q1),
                                   pl.ds(c * sub, sub)],
                    p, c, 0, q1, dims[p][0])

        for c in range(N_SUB):
            for p in range(N_PARTS):
                base = p * part
                cols = pl.ds(c * sub, sub)
                rd[p][c].wait_recv()
                o_s = sr1[p] - kr0[p]
                o_k = kr1[p] - kr0[p]
                partial_ref[pl.ds(base + sr1[p], q2), cols] = (
                    partial_ref[pl.ds(base + sr1[p], q2), cols]
                    + rs_recv[p, c, 0, pl.ds(o_s, q2), :]
                )
                rd[p][c] = start_rs(
                    partial_ref.at[pl.ds(base + sr1[p], q2), cols],
                    p, c, 1, q2, dims[p][1])
                partial_ref[pl.ds(base + kr1[p], q2), cols] = (
                    partial_ref[pl.ds(base + kr1[p], q2), cols]
                    + rs_recv[p, c, 0, pl.ds(o_k, q2), :]
                )

        for c in range(N_SUB):
            for p in range(N_PARTS):
                base = p * part
                cols = pl.ds(c * sub, sub)
                rd[p][c].wait_recv()
                partial_ref[pl.ds(base + kr1[p], q2), cols] = (
                    partial_ref[pl.ds(base + kr1[p], q2), cols]
                    + rs_recv[p, c, 1, pl.ds(0, q2), :]
                )
                rd[p][c] = start_rs(
                    partial_ref.at[pl.ds(base + kr1[p], q2), cols],
                    p, c, 2, q2, dims[p][2])

        for c in range(N_SUB):
            for p in range(N_PARTS):
                base = p * part
                cols = pl.ds(c * sub, sub)
                rd[p][c].wait_recv()
                chunk = jnp.maximum(
                    partial_ref[pl.ds(base + kr1[p], q2), cols]
                    + rs_recv[p, c, 2, pl.ds(0, q2), :],
                    0.0,
                )
                ag_buf[p, c, pl.ds(kr1[p], q2), :] = chunk
                out_ref[pl.ds(base + kr1[p], q2), cols] = chunk.astype(
                    jnp.float32)
                rd[p][c] = start_ag(p, c, 0, kr1[p], q2, dims[p][1])

        for c in range(N_SUB):
            for p in range(N_PARTS):
                base = p * part
                cols = pl.ds(c * sub, sub)
                rd[p][c].wait_recv()
                rd[p][c] = start_ag(p, c, 1, kr0[p], q1, dims[p][0])
                out_ref[pl.ds(base + sr1[p], q2), cols] = ag_buf[
                    p, c, pl.ds(sr1[p], q2), :
                ].astype(jnp.float32)

        for c in range(N_SUB):
            for p in range(N_PARTS):
                base = p * part
                cols = pl.ds(c * sub, sub)
                rd[p][c].wait_recv()
                out_ref[pl.ds(base + sr0[p], q1), cols] = ag_buf[
                    p, c, pl.ds(sr0[p], q1), :
                ].astype(jnp.float32)

        for r in all_rdmas:
            r.wait_send()

    return pl.pallas_call(
        body,
        out_shape=jax.ShapeDtypeStruct((m, n), jnp.float32),
        in_specs=[
            pl.BlockSpec(memory_space=pltpu.VMEM),
            pl.BlockSpec(memory_space=pltpu.VMEM),
        ],
        out_specs=pl.BlockSpec(memory_space=pltpu.VMEM),
        scratch_shapes=[
            pltpu.VMEM((m, n), jnp.bfloat16),
            pltpu.VMEM((N_PARTS, N_SUB, 3, q1, sub), jnp.bfloat16),
            pltpu.VMEM((N_PARTS, N_SUB, part, sub), jnp.bfloat16),
            pltpu.SemaphoreType.DMA((N_PARTS, N_SUB, 3)),
            pltpu.SemaphoreType.DMA((N_PARTS, N_SUB, 3)),
            pltpu.SemaphoreType.DMA((N_PARTS, N_SUB, 2)),
            pltpu.SemaphoreType.DMA((N_PARTS, N_SUB, 2)),
        ],
        compiler_params=pltpu.CompilerParams(
            collective_id=0,
            vmem_limit_bytes=100 * 1024 * 1024,
        ),
    )(A, B)
